# Initial kernel scaffold; baseline (speedup 1.0000x reference)
import jax
import jax.numpy as jnp
from jax.experimental import pallas as pl


def kernel(edge_type, x, edge_index, edge_w, W_rel, W_root, bias):
    raise NotImplementedError("write your pallas kernel here")


# speedup vs baseline: 4.2959x; 4.2959x over previous
import jax
import jax.numpy as jnp
from jax import lax
from jax.experimental import pallas as pl
from jax.experimental.pallas import tpu as pltpu
from jax.experimental.pallas import tpu_sc as plsc

N = 10000
E = 320000
D = 128
NUM_REL = 8
NUM_BASES = 2

NSUB = 16
NCORE = 2
CHUNK = 128
CPS = -(-E // (NSUB * CHUNK))
EPS = CPS * CHUNK
EPAD = NSUB * EPS
ROWS_PER_SUB = 640
N_ACC = NSUB * ROWS_PER_SUB
TBL = NUM_BASES * NUM_REL * N


def _scale_body(w_ref, x_ref, o_ref):
    o_ref[:] = w_ref[pl.program_id(0)] * x_ref[:]


def _build_table(w_flat, x):
    return pl.pallas_call(
        _scale_body,
        grid=(NUM_BASES * NUM_REL,),
        in_specs=[
            pl.BlockSpec(memory_space=pltpu.SMEM),
            pl.BlockSpec((N, D), lambda i: (0, 0)),
        ],
        out_specs=pl.BlockSpec((N, D), lambda i: (i, 0)),
        out_shape=jax.ShapeDtypeStruct((TBL, D), jnp.float32),
    )(w_flat, x)


def _sc_body(table, gidx, dst, zeros, out, gix_v, dst_v, rows_v, acc, sem):
    c = lax.axis_index("c")
    s = lax.axis_index("s")
    base = s * ROWS_PER_SUB

    pltpu.sync_copy(zeros, rows_v)
    for t in range(ROWS_PER_SUB // CHUNK):
        pltpu.sync_copy(rows_v, acc.at[pl.ds(base + t * CHUNK, CHUNK)])
    plsc.subcore_barrier()

    def chunk_step(j, carry):
        pltpu.sync_copy(gidx.at[c, s, j], gix_v)
        pltpu.sync_copy(dst.at[s, j], dst_v)
        pltpu.async_copy(table.at[gix_v], rows_v, sem).wait()
        pltpu.sync_copy(rows_v, acc.at[dst_v], add=True)
        return carry

    lax.fori_loop(0, CPS, chunk_step, 0)
    plsc.subcore_barrier()

    for t in range(ROWS_PER_SUB // CHUNK):
        r = base + t * CHUNK
        pltpu.sync_copy(acc.at[pl.ds(r, CHUNK)], rows_v)
        pltpu.sync_copy(rows_v, out.at[c, pl.ds(r, CHUNK)])


def _sc_aggregate(table, gidx_p, dst_p, zeros):
    mesh = plsc.VectorSubcoreMesh(core_axis_name="c", subcore_axis_name="s")
    fn = pl.kernel(
        _sc_body,
        mesh=mesh,
        out_type=jax.ShapeDtypeStruct((NCORE, N_ACC, D), jnp.float32),
        scratch_types=[
            pltpu.VMEM((CHUNK,), jnp.int32),
            pltpu.VMEM((CHUNK,), jnp.int32),
            pltpu.VMEM((CHUNK, D), jnp.float32),
            pltpu.VMEM_SHARED((N_ACC, D), jnp.float32),
            pltpu.SemaphoreType.DMA,
        ],
    )
    return fn(table, gidx_p, dst_p, zeros)


def _update_body(a0, a1, x_ref, wrel, wroot, bias, o_ref):
    wr = wroot[0] + wroot[1]
    o_ref[:] = (
        jnp.dot(a0[:], wrel[0], preferred_element_type=jnp.float32)
        + jnp.dot(a1[:], wrel[1], preferred_element_type=jnp.float32)
        + jnp.dot(x_ref[:], wr, preferred_element_type=jnp.float32)
        + (bias[0] + bias[1])[None, :]
    )


def _dense_update(a0, a1, x, W_rel, W_root, bias):
    BR = 400
    return pl.pallas_call(
        _update_body,
        grid=(N // BR,),
        in_specs=[
            pl.BlockSpec((BR, D), lambda i: (i, 0)),
            pl.BlockSpec((BR, D), lambda i: (i, 0)),
            pl.BlockSpec((BR, D), lambda i: (i, 0)),
            pl.BlockSpec((NUM_BASES, D, D), lambda i: (0, 0, 0)),
            pl.BlockSpec((NUM_BASES, D, D), lambda i: (0, 0, 0)),
            pl.BlockSpec((NUM_BASES, D), lambda i: (0, 0)),
        ],
        out_specs=pl.BlockSpec((BR, D), lambda i: (i, 0)),
        out_shape=jax.ShapeDtypeStruct((N, D), jnp.float32),
    )(a0, a1, x, W_rel, W_root, bias)


def kernel(edge_type, x, edge_index, edge_w, W_rel, W_root, bias):
    src = edge_index[0].astype(jnp.int32)
    dst = edge_index[1].astype(jnp.int32)
    et = edge_type.astype(jnp.int32)

    gidx = et * N + src
    gidx2 = jnp.stack([gidx, gidx + NUM_REL * N])
    pad = EPAD - E
    gidx_p = jnp.pad(gidx2, ((0, 0), (0, pad))).reshape(NCORE, NSUB, CPS, CHUNK)
    dst_p = jnp.pad(dst, (0, pad), constant_values=N).reshape(NSUB, CPS, CHUNK)

    w_flat = edge_w.reshape(NUM_BASES * NUM_REL)
    table = _build_table(w_flat, x)
    zeros = jnp.zeros((CHUNK, D), jnp.float32)

    aggr = _sc_aggregate(table, gidx_p, dst_p, zeros)
    a0 = aggr[0, :N]
    a1 = aggr[1, :N]
    return _dense_update(a0, a1, x, W_rel, W_root, bias)
